# baseline (speedup 1.0000x reference)
import jax
import jax.numpy as jnp
from jax import lax
from jax.experimental import pallas as pl
from jax.experimental.pallas import tpu as pltpu
from jax.experimental.pallas import tpu_sc as plsc

B = 1024
EMB = 32
LANES = 128
NC = 2
NS = 16
NW = NC * NS
BPW = B // NW
SB = 4
NSB = BPW // SB
SBG = 16 // SB


def _extract(chunk16, lane):
    mask = lax.iota(jnp.int32, 16) == lane
    return jnp.max(jnp.where(mask, chunk16, jnp.int32(-2147483648)))


def _sc_body(uid_hbm, iid_hbm, uet_hbm, iet_hbm, ubt_hbm, ibt_hbm, utt_hbm,
             y_hbm, t_hbm,
             uid_v, iid_v, ucol_v, icol_v, ustr_v, istr_v,
             ubs_v, ibs_v, uts_v, prod_v, bsum_v, y_v, t_v, sem0, sem1):
    wid = lax.axis_index("s") * NC + lax.axis_index("c")
    base = wid * BPW
    sems = (sem0, sem1)

    pltpu.sync_copy(uid_hbm.at[pl.ds(base, BPW)], uid_v)
    pltpu.sync_copy(iid_hbm.at[pl.ds(base, BPW)], iid_v)

    lanes = lax.iota(jnp.int32, 16)
    zeros = jnp.zeros((16,), jnp.int32)
    for c in range(BPW // 16):
        s = pl.ds(c * 16, 16)
        ucol_v[s] = uid_v[s] % LANES
        icol_v[s] = iid_v[s] % LANES

    def fire(sb):
        g = sb // SBG
        q = sb % SBG
        buf = sb % 2
        sem = sems[buf]
        uchunk = uid_v[pl.ds(g * 16, 16)]
        ichunk = iid_v[pl.ds(g * 16, 16)]
        ucols = []
        icols = []
        copies = []
        for k in range(SB):
            lane = q * SB + k
            uidk = _extract(uchunk, lane)
            iidk = _extract(ichunk, lane)
            ustripe = (uidk // LANES) * LANES
            istripe = (iidk // LANES) * LANES
            ucols.append(uidk % LANES)
            icols.append(iidk % LANES)
            copies.append(pltpu.async_copy(
                uet_hbm.at[:, pl.ds(ustripe, LANES)], ustr_v.at[buf, k], sem))
            copies.append(pltpu.async_copy(
                iet_hbm.at[:, pl.ds(istripe, LANES)], istr_v.at[buf, k], sem))
            copies.append(pltpu.async_copy(
                ubt_hbm.at[:, pl.ds(ustripe, LANES)], ubs_v.at[lane], sem))
            copies.append(pltpu.async_copy(
                ibt_hbm.at[:, pl.ds(istripe, LANES)], ibs_v.at[lane], sem))
            copies.append(pltpu.async_copy(
                utt_hbm.at[:, pl.ds(ustripe, LANES)], uts_v.at[lane], sem))
        return (sb, copies, ucols, icols)

    def drain_extract(st):
        sb, copies, ucols, icols = st
        g = sb // SBG
        buf = sb % 2
        for cc in copies:
            cc.wait()
        for k in range(SB):
            bk16 = jnp.zeros((16,), jnp.int32) + buf
            k16 = jnp.zeros((16,), jnp.int32) + k
            cu = jnp.zeros((16,), jnp.int32) + ucols[k]
            ci = jnp.zeros((16,), jnp.int32) + icols[k]
            u_lo = plsc.load_gather(ustr_v, [bk16, k16, lanes, cu])
            u_hi = plsc.load_gather(ustr_v, [bk16, k16, lanes + 16, cu])
            i_lo = plsc.load_gather(istr_v, [bk16, k16, lanes, ci])
            i_hi = plsc.load_gather(istr_v, [bk16, k16, lanes + 16, ci])
            prod_v[pl.ds((sb * SB + k) * 16, 16)] = (
                u_lo * i_lo + u_hi * i_hi)
        if sb % SBG == SBG - 1:
            sg = pl.ds(g * 16, 16)
            cu16 = ucol_v[sg]
            ci16 = icol_v[sg]
            bsum_v[sg] = (plsc.load_gather(ubs_v, [lanes, zeros, cu16])
                          + plsc.load_gather(ibs_v, [lanes, zeros, ci16]))
            t_v[sg] = plsc.load_gather(uts_v, [lanes, zeros, cu16])

    prev = fire(0)
    for sb in range(1, NSB):
        cur = fire(sb)
        drain_extract(prev)
        prev = cur
    drain_extract(prev)

    for g in range(BPW // 16):
        s = pl.ds(g * 16, 16)
        rowbase = (g * 16 + lanes) * 16
        acc = bsum_v[s]
        for l in range(16):
            acc = acc + plsc.load_gather(prod_v, [rowbase + l])
        y_v[s] = acc

    pltpu.sync_copy(y_v, y_hbm.at[pl.ds(base, BPW)])
    pltpu.sync_copy(t_v, t_hbm.at[pl.ds(base, BPW)])


def _tc_body(y_ref, t_ref, out_ref):
    out_ref[...] = 1.0 / (1.0 + jnp.exp(y_ref[...] - t_ref[...]))


@jax.jit
def _impl(user_ids, item_ids, user_embeddings, item_embeddings,
          user_biases, item_biases, user_ts):
    uids = user_ids.astype(jnp.int32)
    iids = item_ids.astype(jnp.int32)
    mesh = plsc.VectorSubcoreMesh(core_axis_name="c", subcore_axis_name="s")
    y, t = pl.kernel(
        _sc_body,
        out_type=(
            jax.ShapeDtypeStruct((B,), jnp.float32),
            jax.ShapeDtypeStruct((B,), jnp.float32),
        ),
        mesh=mesh,
        scratch_types=[
            pltpu.VMEM((BPW,), jnp.int32),
            pltpu.VMEM((BPW,), jnp.int32),
            pltpu.VMEM((BPW,), jnp.int32),
            pltpu.VMEM((BPW,), jnp.int32),
            pltpu.VMEM((2, SB, EMB, LANES), jnp.float32),
            pltpu.VMEM((2, SB, EMB, LANES), jnp.float32),
            pltpu.VMEM((16, 1, LANES), jnp.float32),
            pltpu.VMEM((16, 1, LANES), jnp.float32),
            pltpu.VMEM((16, 1, LANES), jnp.float32),
            pltpu.VMEM((BPW * 16,), jnp.float32),
            pltpu.VMEM((BPW,), jnp.float32),
            pltpu.VMEM((BPW,), jnp.float32),
            pltpu.VMEM((BPW,), jnp.float32),
            pltpu.SemaphoreType.DMA,
            pltpu.SemaphoreType.DMA,
        ],
        compiler_params=pltpu.CompilerParams(needs_layout_passes=False,
                                             use_tc_tiling_on_sc=True),
    )(uids, iids,
      user_embeddings.T, item_embeddings.T,
      user_biases.T, item_biases.T, user_ts.T)

    rows = 1024
    return pl.pallas_call(
        _tc_body,
        grid=(B // rows,),
        in_specs=[
            pl.BlockSpec((1, B), lambda i: (0, 0)),
            pl.BlockSpec((rows, 1), lambda i: (i, 0)),
        ],
        out_specs=pl.BlockSpec((rows, B), lambda i: (i, 0)),
        out_shape=jax.ShapeDtypeStruct((B, B), jnp.float32),
    )(y.reshape(1, B), t.reshape(B, 1))


def kernel(user_ids, item_ids, user_embeddings, item_embeddings,
           user_biases, item_biases, user_ts, user_betas):
    del user_betas
    return _impl(user_ids, item_ids, user_embeddings, item_embeddings,
                 user_biases, item_biases, user_ts)


# speedup vs baseline: 1.0120x; 1.0120x over previous
import jax
import jax.numpy as jnp
from jax import lax
from jax.experimental import pallas as pl
from jax.experimental.pallas import tpu as pltpu
from jax.experimental.pallas import tpu_sc as plsc

B = 1024
EMB = 32
LANES = 128
NC = 2
NS = 16
NW = NC * NS
BPW = B // NW
SB = 4
NSB = BPW // SB
SBG = 16 // SB


def _extract(chunk16, lane):
    mask = lax.iota(jnp.int32, 16) == lane
    return jnp.max(jnp.where(mask, chunk16, jnp.int32(-2147483648)))


def _sc_body(uid_hbm, iid_hbm, uet_hbm, iet_hbm, ubt_hbm, ibt_hbm, utt_hbm,
             y_hbm, t_hbm,
             uid_v, iid_v, ucol_v, icol_v, ustr_v, istr_v,
             ubs_v, ibs_v, uts_v, prod_v, bsum_v, y_v, t_v, sem0, sem1):
    wid = lax.axis_index("s") * NC + lax.axis_index("c")
    base = wid * BPW
    sems = (sem0, sem1)

    pltpu.sync_copy(uid_hbm.at[pl.ds(base, BPW)], uid_v)
    pltpu.sync_copy(iid_hbm.at[pl.ds(base, BPW)], iid_v)

    lanes = lax.iota(jnp.int32, 16)
    zeros = jnp.zeros((16,), jnp.int32)
    for c in range(BPW // 16):
        s = pl.ds(c * 16, 16)
        ucol_v[s] = uid_v[s] % LANES
        icol_v[s] = iid_v[s] % LANES

    def fire(sb):
        g = sb // SBG
        q = sb % SBG
        buf = sb % 2
        sem = sems[buf]
        uchunk = uid_v[pl.ds(g * 16, 16)]
        ichunk = iid_v[pl.ds(g * 16, 16)]
        ucols = []
        icols = []
        copies = []
        for k in range(SB):
            lane = q * SB + k
            uidk = _extract(uchunk, lane)
            iidk = _extract(ichunk, lane)
            ustripe = (uidk // LANES) * LANES
            istripe = (iidk // LANES) * LANES
            ucols.append(uidk % LANES)
            icols.append(iidk % LANES)
            copies.append(pltpu.async_copy(
                uet_hbm.at[:, pl.ds(ustripe, LANES)], ustr_v.at[buf, k], sem))
            copies.append(pltpu.async_copy(
                iet_hbm.at[:, pl.ds(istripe, LANES)], istr_v.at[buf, k], sem))
            copies.append(pltpu.async_copy(
                ubt_hbm.at[:, pl.ds(ustripe, LANES)], ubs_v.at[lane], sem))
            copies.append(pltpu.async_copy(
                ibt_hbm.at[:, pl.ds(istripe, LANES)], ibs_v.at[lane], sem))
            copies.append(pltpu.async_copy(
                utt_hbm.at[:, pl.ds(ustripe, LANES)], uts_v.at[lane], sem))
        return (sb, copies, ucols, icols)

    def drain_extract(st):
        sb, copies, ucols, icols = st
        g = sb // SBG
        buf = sb % 2
        for cc in copies:
            cc.wait()
        for k in range(SB):
            bk16 = jnp.zeros((16,), jnp.int32) + buf
            k16 = jnp.zeros((16,), jnp.int32) + k
            cu = jnp.zeros((16,), jnp.int32) + ucols[k]
            ci = jnp.zeros((16,), jnp.int32) + icols[k]
            u_lo = plsc.load_gather(ustr_v, [bk16, k16, lanes, cu])
            u_hi = plsc.load_gather(ustr_v, [bk16, k16, lanes + 16, cu])
            i_lo = plsc.load_gather(istr_v, [bk16, k16, lanes, ci])
            i_hi = plsc.load_gather(istr_v, [bk16, k16, lanes + 16, ci])
            prod_v[pl.ds((sb * SB + k) * 16, 16)] = (
                u_lo * i_lo + u_hi * i_hi)
        if sb % SBG == SBG - 1:
            sg = pl.ds(g * 16, 16)
            cu16 = ucol_v[sg]
            ci16 = icol_v[sg]
            bsum_v[sg] = (plsc.load_gather(ubs_v, [lanes, zeros, cu16])
                          + plsc.load_gather(ibs_v, [lanes, zeros, ci16]))
            t_v[sg] = plsc.load_gather(uts_v, [lanes, zeros, cu16])

    prev = fire(0)
    for sb in range(1, NSB):
        cur = fire(sb)
        drain_extract(prev)
        prev = cur
    drain_extract(prev)

    for g in range(BPW // 16):
        s = pl.ds(g * 16, 16)
        rowbase = (g * 16 + lanes) * 16
        acc = bsum_v[s]
        for l in range(16):
            acc = acc + plsc.load_gather(prod_v, [rowbase + l])
        y_v[s] = acc

    pltpu.sync_copy(y_v, y_hbm.at[pl.ds(base, BPW)])
    pltpu.sync_copy(t_v, t_hbm.at[pl.ds(base, BPW)])


def _tc_body(y_ref, t_ref, out_ref):
    out_ref[...] = 1.0 / (1.0 + jnp.exp(y_ref[...] - t_ref[...]))


@jax.jit
def _impl(user_ids, item_ids, user_embeddings, item_embeddings,
          user_biases, item_biases, user_ts):
    uids = user_ids.astype(jnp.int32)
    iids = item_ids.astype(jnp.int32)
    mesh = plsc.VectorSubcoreMesh(core_axis_name="c", subcore_axis_name="s")
    y, t = pl.kernel(
        _sc_body,
        out_type=(
            jax.ShapeDtypeStruct((B,), jnp.float32),
            jax.ShapeDtypeStruct((B,), jnp.float32),
        ),
        mesh=mesh,
        scratch_types=[
            pltpu.VMEM((BPW,), jnp.int32),
            pltpu.VMEM((BPW,), jnp.int32),
            pltpu.VMEM((BPW,), jnp.int32),
            pltpu.VMEM((BPW,), jnp.int32),
            pltpu.VMEM((2, SB, EMB, LANES), jnp.float32),
            pltpu.VMEM((2, SB, EMB, LANES), jnp.float32),
            pltpu.VMEM((16, 1, LANES), jnp.float32),
            pltpu.VMEM((16, 1, LANES), jnp.float32),
            pltpu.VMEM((16, 1, LANES), jnp.float32),
            pltpu.VMEM((BPW * 16,), jnp.float32),
            pltpu.VMEM((BPW,), jnp.float32),
            pltpu.VMEM((BPW,), jnp.float32),
            pltpu.VMEM((BPW,), jnp.float32),
            pltpu.SemaphoreType.DMA,
            pltpu.SemaphoreType.DMA,
        ],
        compiler_params=pltpu.CompilerParams(needs_layout_passes=False,
                                             use_tc_tiling_on_sc=True),
    )(uids, iids,
      user_embeddings.T, item_embeddings.T,
      user_biases.T, item_biases.T, user_ts.T)

    rows = 512
    return pl.pallas_call(
        _tc_body,
        grid=(B // rows,),
        in_specs=[
            pl.BlockSpec((1, B), lambda i: (0, 0)),
            pl.BlockSpec((rows, 1), lambda i: (i, 0)),
        ],
        out_specs=pl.BlockSpec((rows, B), lambda i: (i, 0)),
        out_shape=jax.ShapeDtypeStruct((B, B), jnp.float32),
    )(y.reshape(1, B), t.reshape(B, 1))


def kernel(user_ids, item_ids, user_embeddings, item_embeddings,
           user_biases, item_biases, user_ts, user_betas):
    del user_betas
    return _impl(user_ids, item_ids, user_embeddings, item_embeddings,
                 user_biases, item_biases, user_ts)
